# baseline (speedup 1.0000x reference)
import jax
import jax.numpy as jnp
from jax import lax
from jax.experimental import pallas as pl
from jax.experimental.pallas import tpu as pltpu
from jax.experimental.pallas import tpu_sc as plsc

VOCAB = 1000000
D_MODEL = 64

_info = plsc.get_sparse_core_info()
_NC, _NS = _info.num_cores, _info.num_subcores
_NW = _NC * _NS

_S = 4096
_T = 200
_TP = 208
_SPW = _S // _NW
_NBUF = 2
_ROUNDS = _SPW // _NBUF
_LANES = 16
_NVEC = _TP // _LANES


def _embed_kernel(x_hbm, table_hbm, out_hbm,
                  idx0, idx1, k0, k1, pair_v, comp_v, gsem, wsem):
    wid = lax.axis_index("s") * _NC + lax.axis_index("c")
    tok_base = wid * _SPW * _T

    idx_b = [idx0, idx1]
    k_b = [k0, k1]

    def prep(c, b):
        idx_b[b][pl.ds(_T - 8, 16)] = jnp.zeros((16,), jnp.int32)
        pltpu.sync_copy(x_hbm.at[pl.ds(tok_base + c * _T, _T)],
                        idx_b[b].at[pl.ds(0, _T)])
        for t in range(_NVEC):
            v = idx_b[b][pl.ds(t * _LANES, _LANES)]
            k_b[b][pl.ds(t * _LANES, _LANES)] = lax.shift_right_logical(v, 1)

    def gather_start(b):
        pltpu.async_copy(table_hbm.at[k_b[b]], pair_v.at[b], gsem.at[b])

    def gather_wait(b):
        pltpu.make_async_copy(table_hbm.at[k_b[b]], pair_v.at[b],
                              gsem.at[b]).wait()

    def compact(b):
        pv = pair_v.at[b]
        cv = comp_v.at[b]
        for g in range(_NVEC):
            pos = lax.iota(jnp.int32, _LANES) + (g * _LANES)
            v = idx_b[b][pl.ds(g * _LANES, _LANES)]
            halfoff = (v & 1) * D_MODEL
            for u in range(D_MODEL):
                val = plsc.load_gather(pv, [pos, halfoff + u])
                plsc.store_scatter(cv, [pos, jnp.full((_LANES,), u,
                                                      jnp.int32)], val)

    def wb_start(c, b):
        pltpu.async_copy(comp_v.at[b, pl.ds(0, _T)],
                         out_hbm.at[pl.ds(tok_base + c * _T, _T)],
                         wsem.at[b])

    def wb_wait(c, b):
        pltpu.make_async_copy(comp_v.at[b, pl.ds(0, _T)],
                              out_hbm.at[pl.ds(tok_base + c * _T, _T)],
                              wsem.at[b]).wait()

    for b in range(_NBUF):
        prep(b, b)
        gather_start(b)

    def round_body(r, carry):
        for b in range(_NBUF):
            c = r * _NBUF + b
            gather_wait(b)
            compact(b)
            wb_start(c, b)
            wb_wait(c, b)

            @pl.when(c + _NBUF < _SPW)
            def _():
                prep(c + _NBUF, b)
                gather_start(b)

        return carry

    lax.fori_loop(0, _ROUNDS, round_body, 0)


@jax.jit
def kernel(x, table):
    mesh = plsc.VectorSubcoreMesh(core_axis_name="c", subcore_axis_name="s")
    out = pl.kernel(
        _embed_kernel,
        mesh=mesh,
        out_type=jax.ShapeDtypeStruct((_S * _T, D_MODEL), jnp.float32),
        scratch_types=[
            pltpu.VMEM((_TP,), jnp.int32),
            pltpu.VMEM((_TP,), jnp.int32),
            pltpu.VMEM((_TP,), jnp.int32),
            pltpu.VMEM((_TP,), jnp.int32),
            pltpu.VMEM((_NBUF, _TP, 2 * D_MODEL), jnp.float32),
            pltpu.VMEM((_NBUF, _TP, D_MODEL), jnp.float32),
            pltpu.SemaphoreType.DMA((_NBUF,)),
            pltpu.SemaphoreType.DMA((_NBUF,)),
        ],
        compiler_params=pltpu.CompilerParams(use_tc_tiling_on_sc=True,
                                             needs_layout_passes=False),
    )(x.reshape(-1).astype(jnp.int32),
      table.reshape(VOCAB // 2, 2 * D_MODEL))
    return out.reshape(_S, _T, D_MODEL)


# speedup vs baseline: 1.0346x; 1.0346x over previous
import jax
import jax.numpy as jnp
from jax import lax
from jax.experimental import pallas as pl
from jax.experimental.pallas import tpu as pltpu
from jax.experimental.pallas import tpu_sc as plsc

VOCAB = 1000000
D_MODEL = 64

_info = plsc.get_sparse_core_info()
_NC, _NS = _info.num_cores, _info.num_subcores
_NW = _NC * _NS

_S = 4096
_T = 200
_TP = 208
_SPW = _S // _NW
_NBUF = 2
_ROUNDS = _SPW // _NBUF
_LANES = 16
_NVEC = _TP // _LANES


def _embed_kernel(x_hbm, table_hbm, out_hbm,
                  idx0, idx1, k0, k1, pair_v, comp_v, gsem, wsem):
    wid = lax.axis_index("s") * _NC + lax.axis_index("c")
    tok_base = wid * _SPW * _T

    idx_b = [idx0, idx1]
    k_b = [k0, k1]

    def prep(c, b):
        idx_b[b][pl.ds(_T - 8, 16)] = jnp.zeros((16,), jnp.int32)
        pltpu.sync_copy(x_hbm.at[pl.ds(tok_base + c * _T, _T)],
                        idx_b[b].at[pl.ds(0, _T)])
        for t in range(_NVEC):
            v = idx_b[b][pl.ds(t * _LANES, _LANES)]
            k_b[b][pl.ds(t * _LANES, _LANES)] = lax.shift_right_logical(v, 1)

    def gather_start(b):
        pltpu.async_copy(table_hbm.at[k_b[b]], pair_v.at[b], gsem.at[b])

    def gather_wait(b):
        pltpu.make_async_copy(table_hbm.at[k_b[b]], pair_v.at[b],
                              gsem.at[b]).wait()

    def compact(b):
        pv = pair_v.at[b]
        cv = comp_v.at[b]
        zero = jnp.zeros((_LANES,), jnp.int32)
        for g in range(_NVEC):
            pos = lax.iota(jnp.int32, _LANES) + (g * _LANES)
            v = idx_b[b][pl.ds(g * _LANES, _LANES)]
            halfoff = (v & 1) * D_MODEL

            @plsc.parallel_loop(0, D_MODEL, unroll=8)
            def _(u):
                val = plsc.load_gather(pv, [pos, halfoff + u])
                plsc.store_scatter(cv, [pos, zero + u], val)

    def wb_start(c, b):
        pltpu.async_copy(comp_v.at[b, pl.ds(0, _T)],
                         out_hbm.at[pl.ds(tok_base + c * _T, _T)],
                         wsem.at[b])

    def wb_wait(c, b):
        pltpu.make_async_copy(comp_v.at[b, pl.ds(0, _T)],
                              out_hbm.at[pl.ds(tok_base + c * _T, _T)],
                              wsem.at[b]).wait()

    for b in range(_NBUF):
        prep(b, b)
        gather_start(b)

    def round_body(r, carry):
        for b in range(_NBUF):
            c = r * _NBUF + b
            gather_wait(b)
            compact(b)
            wb_start(c, b)
            wb_wait(c, b)

            @pl.when(c + _NBUF < _SPW)
            def _():
                prep(c + _NBUF, b)
                gather_start(b)

        return carry

    lax.fori_loop(0, _ROUNDS, round_body, 0)


@jax.jit
def kernel(x, table):
    mesh = plsc.VectorSubcoreMesh(core_axis_name="c", subcore_axis_name="s")
    out = pl.kernel(
        _embed_kernel,
        mesh=mesh,
        out_type=jax.ShapeDtypeStruct((_S * _T, D_MODEL), jnp.float32),
        scratch_types=[
            pltpu.VMEM((_TP,), jnp.int32),
            pltpu.VMEM((_TP,), jnp.int32),
            pltpu.VMEM((_TP,), jnp.int32),
            pltpu.VMEM((_TP,), jnp.int32),
            pltpu.VMEM((_NBUF, _TP, 2 * D_MODEL), jnp.float32),
            pltpu.VMEM((_NBUF, _TP, D_MODEL), jnp.float32),
            pltpu.SemaphoreType.DMA((_NBUF,)),
            pltpu.SemaphoreType.DMA((_NBUF,)),
        ],
        compiler_params=pltpu.CompilerParams(use_tc_tiling_on_sc=True,
                                             needs_layout_passes=False),
    )(x.reshape(-1).astype(jnp.int32),
      table.reshape(VOCAB // 2, 2 * D_MODEL))
    return out.reshape(_S, _T, D_MODEL)


# speedup vs baseline: 2.3775x; 2.2980x over previous
import jax
import jax.numpy as jnp
from jax import lax
from jax.experimental import pallas as pl
from jax.experimental.pallas import tpu as pltpu
from jax.experimental.pallas import tpu_sc as plsc

VOCAB = 1000000
D_MODEL = 64

_info = plsc.get_sparse_core_info()
_NC, _NS = _info.num_cores, _info.num_subcores
_NW = _NC * _NS

_S = 4096
_T = 200
_SPW = _S // _NW
_NBUF = 4
_ROUNDS = _SPW // _NBUF


def _embed_kernel(x_hbm, table_hbm, out_hbm, idx_v, rows_v, gsem, wsem):
    wid = lax.axis_index("s") * _NC + lax.axis_index("c")
    s_base = wid * _SPW

    pltpu.sync_copy(x_hbm.at[pl.ds(s_base, _SPW)], idx_v)

    def gather_start(c, b):
        pltpu.async_copy(table_hbm.at[idx_v.at[c]], rows_v.at[b, 0],
                         gsem.at[b])

    def gather_wait(c, b):
        pltpu.make_async_copy(table_hbm.at[idx_v.at[c]], rows_v.at[b, 0],
                              gsem.at[b]).wait()

    def wb_start(c, b):
        pltpu.async_copy(rows_v.at[b], out_hbm.at[pl.ds(s_base + c, 1)],
                         wsem.at[b])

    def wb_wait(c, b):
        pltpu.make_async_copy(rows_v.at[b], out_hbm.at[pl.ds(s_base + c, 1)],
                              wsem.at[b]).wait()

    for b in range(_NBUF):
        gather_start(b, b)

    def round_body(r, carry):
        for b in range(_NBUF):
            c = r * _NBUF + b
            gather_wait(c, b)
            wb_start(c, b)
            wb_wait(c, b)
            gather_start(c + _NBUF, b)
        return carry

    lax.fori_loop(0, _ROUNDS - 1, round_body, 0)

    for b in range(_NBUF):
        c = (_ROUNDS - 1) * _NBUF + b
        gather_wait(c, b)
        wb_start(c, b)
        wb_wait(c, b)


@jax.jit
def kernel(x, table):
    mesh = plsc.VectorSubcoreMesh(core_axis_name="c", subcore_axis_name="s")
    out = pl.kernel(
        _embed_kernel,
        mesh=mesh,
        out_type=jax.ShapeDtypeStruct((_S, _T, D_MODEL), jnp.float32),
        scratch_types=[
            pltpu.VMEM((_SPW, _T), jnp.int32),
            pltpu.VMEM((_NBUF, 1, _T, D_MODEL), jnp.float32),
            pltpu.SemaphoreType.DMA((_NBUF,)),
            pltpu.SemaphoreType.DMA((_NBUF,)),
        ],
        compiler_params=pltpu.CompilerParams(use_tc_tiling_on_sc=False),
    )(x.astype(jnp.int32), table)
    return out
